# Initial kernel scaffold; baseline (speedup 1.0000x reference)
import jax
import jax.numpy as jnp
from jax.experimental import pallas as pl


def kernel(x, edge_index, W1, b1, W2, b2, lstm_Wx, lstm_bx, lstm_Wh, lstm_bh, lstm_wc, lstm_b, Wl, bl):
    raise NotImplementedError("write your pallas kernel here")


# speedup vs baseline: 3.8128x; 3.8128x over previous
import functools

import jax
import jax.numpy as jnp
from jax import lax
from jax.experimental import pallas as pl
from jax.experimental.pallas import tpu as pltpu
from jax.experimental.pallas import tpu_sc as plsc

D = 128
CHUNK = 128
NCORES = 2
NSUB = 16
NTILES = NCORES * NSUB
HISTW = 16

_sc_mesh = plsc.VectorSubcoreMesh(core_axis_name="c", subcore_axis_name="s")


def _round_up(a, m):
    return (a + m - 1) // m * m


def _make_edge_kernel(np_, cpt):
    half = np_ // NCORES
    acc_rows = half + CHUNK
    nz = acc_rows // CHUNK
    nw = half // CHUNK
    zper = _round_up(nz, NSUB) // NSUB

    @functools.partial(
        pl.kernel,
        mesh=_sc_mesh,
        out_type=jax.ShapeDtypeStruct((NCORES, half, D), jnp.float32),
        scratch_types=[
            pltpu.VMEM((cpt, CHUNK), jnp.int32),
            pltpu.VMEM((cpt, CHUNK), jnp.int32),
            pltpu.VMEM((CHUNK, D), jnp.float32),
            pltpu.VMEM((CHUNK, D), jnp.float32),
            pltpu.SemaphoreType.DMA,
            pltpu.SemaphoreType.DMA,
            pltpu.VMEM_SHARED((acc_rows, D), jnp.float32),
        ],
    )
    def edge_kernel(y, r2d, cl2d, zrows, out, ridx, cidx, buf0, buf1, g0, g1, acc):
        cid = lax.axis_index("c")
        sid = lax.axis_index("s")
        pltpu.sync_copy(zrows, buf0)
        for z in range(zper):
            zi = z * NSUB + sid
            @pl.when(zi < nz)
            def _():
                pltpu.sync_copy(buf0, acc.at[pl.ds(zi * CHUNK, CHUNK)])
        pltpu.sync_copy(r2d.at[pl.ds(sid * cpt, cpt)], ridx)
        pltpu.sync_copy(cl2d.at[cid, pl.ds(sid * cpt, cpt)], cidx)
        plsc.subcore_barrier()

        def body(j, carry):
            pltpu.make_async_copy(y.at[ridx.at[j]], buf0, g0).start()
            pltpu.make_async_copy(y.at[ridx.at[j]], buf0, g0).wait()
            pltpu.sync_copy(buf0, acc.at[cidx.at[j]], add=True)
            return carry

        lax.fori_loop(0, cpt, body, 0)
        plsc.subcore_barrier()

        for z in range(zper):
            zi = z * NSUB + sid
            @pl.when(zi < nw)
            def _():
                pltpu.sync_copy(acc.at[pl.ds(zi * CHUNK, CHUNK)], buf0)
                pltpu.sync_copy(buf0, out.at[cid, pl.ds(zi * CHUNK, CHUNK)])

    return edge_kernel


def _mm_scaled_body(d_ref, x_ref, w_ref, o_ref):
    o_ref[...] = jnp.dot(d_ref[...] * x_ref[...], w_ref[...],
                         preferred_element_type=jnp.float32)


def _gcn_epilogue_mm_body(s_ref, d_ref, y_ref, b_ref, w_ref, o_ref):
    dis = d_ref[...]
    h = jax.nn.relu(dis * (s_ref[...] + 2.0 * y_ref[...]) + b_ref[...])
    o_ref[...] = jnp.dot(dis * h, w_ref[...], preferred_element_type=jnp.float32)


def _lstm_head_body(s_ref, d_ref, y_ref, b_ref, wi_ref, wt_ref, wo_ref,
                    bi_ref, bt_ref, bo_ref, wc_ref, wl_ref, bl_ref, o_ref):
    dis = d_ref[...]
    X = jax.nn.relu(dis * (s_ref[...] + 2.0 * y_ref[...]) + b_ref[...])
    for k in range(3):
        AI = jnp.dot(X, wi_ref[k], preferred_element_type=jnp.float32) + bi_ref[k]
        AT = jnp.dot(X, wt_ref[k], preferred_element_type=jnp.float32) + bt_ref[k]
        AO = jnp.dot(X, wo_ref[k], preferred_element_type=jnp.float32) + bo_ref[k]
        Cn = jax.nn.sigmoid(AI) * jnp.tanh(AT)
        X = jax.nn.sigmoid(AO + wc_ref[k] * Cn) * jnp.tanh(Cn)
    o_ref[...] = jnp.dot(jax.nn.relu(X), wl_ref[...],
                         preferred_element_type=jnp.float32) + bl_ref[...]


def kernel(x, edge_index, W1, b1, W2, b2, lstm_Wx, lstm_bx, lstm_Wh, lstm_bh,
           lstm_wc, lstm_b, Wl, bl):
    n, d = x.shape
    out_dim = Wl.shape[1]
    e = edge_index.shape[1]
    np_ = _round_up(max(n + 1, NSUB * CHUNK), NSUB * CHUNK)
    half = np_ // NCORES
    cpt_h = _round_up(e, NTILES * CHUNK) // (NTILES * CHUNK)
    cpt_h += cpt_h % 2
    e_pad = cpt_h * NTILES * CHUNK
    cpt_e = e_pad // (NSUB * CHUNK)
    blk = 512
    grid = (np_ // blk,)

    r = edge_index[0]
    c = edge_index[1]
    rp = jnp.concatenate([r, jnp.full((e_pad - e,), np_ - 1, jnp.int32)])
    cp = jnp.concatenate([c, jnp.full((e_pad - e,), n, jnp.int32)])
    r2d = rp.reshape(e_pad // CHUNK, CHUNK)
    c2d = cp.reshape(e_pad // CHUNK, CHUNK)
    trash = half + (jnp.arange(e_pad, dtype=jnp.int32) % CHUNK).reshape(c2d.shape)
    cl = []
    for core in range(NCORES):
        loc = cp.reshape(c2d.shape) - core * half
        inb = (loc >= 0) & (loc < half)
        cl.append(jnp.where(inb, loc, trash))
    cl2d = jnp.stack(cl)
    xp = jnp.zeros((np_, d), x.dtype).at[:n].set(x)
    onespat = jnp.zeros((CHUNK, HISTW), jnp.float32).at[:, 0].set(1.0)
    zhist = jnp.zeros((np_ // NSUB, HISTW), jnp.float32)
    zrows = jnp.zeros((CHUNK, d), jnp.float32)
    b1r = b1[None, :]
    b2r = b2[None, :]
    wi = lstm_Wx[:, 0]
    wt = lstm_Wx[:, 2]
    wo = lstm_Wx[:, 3]
    bi = (lstm_bx[:, 0] + lstm_bh[:, 0] + lstm_b[:, 0])[:, None, :]
    bt = (lstm_bx[:, 2] + lstm_bh[:, 2] + lstm_b[:, 2])[:, None, :]
    bo = (lstm_bx[:, 3] + lstm_bh[:, 3] + lstm_b[:, 3])[:, None, :]
    wco = lstm_wc[:, 3][:, None, :]
    wlp = jnp.zeros((d, D), jnp.float32).at[:, :out_dim].set(Wl)
    blp = jnp.zeros((1, D), jnp.float32).at[0, :out_dim].set(bl)

    edge_call = _make_edge_kernel(np_, cpt_e)

    ones_mat = jnp.ones((np_, d), jnp.float32)
    deg = edge_call(ones_mat, r2d, cl2d, zrows).reshape(np_, d)[:, 0] + 2.0
    dis = jnp.broadcast_to(lax.rsqrt(deg)[:, None], (np_, d))

    row_spec = pl.BlockSpec((blk, d), lambda i: (i, 0))
    w_spec = pl.BlockSpec((d, D), lambda i: (0, 0))
    b_spec = pl.BlockSpec((1, D), lambda i: (0, 0))
    w3_spec = pl.BlockSpec((3, d, d), lambda i: (0, 0, 0))
    b3_spec = pl.BlockSpec((3, 1, d), lambda i: (0, 0, 0))
    rowD = jax.ShapeDtypeStruct((np_, D), jnp.float32)

    y1 = pl.pallas_call(
        _mm_scaled_body, grid=grid,
        in_specs=[row_spec, row_spec, w_spec],
        out_specs=row_spec, out_shape=rowD,
    )(dis, xp, W1)
    seg1 = edge_call(y1, r2d, cl2d, zrows).reshape(np_, d)

    y2 = pl.pallas_call(
        _gcn_epilogue_mm_body, grid=grid,
        in_specs=[row_spec, row_spec, row_spec, b_spec, w_spec],
        out_specs=row_spec, out_shape=rowD,
    )(seg1, dis, y1, b1r, W2)
    seg2 = edge_call(y2, r2d, cl2d, zrows).reshape(np_, d)

    o = pl.pallas_call(
        _lstm_head_body, grid=grid,
        in_specs=[row_spec, row_spec, row_spec, b_spec,
                  w3_spec, w3_spec, w3_spec, b3_spec, b3_spec, b3_spec,
                  b3_spec, w_spec, b_spec],
        out_specs=row_spec, out_shape=rowD,
    )(seg2, dis, y2, b2r, wi, wt, wo, bi, bt, bo, wco, wlp, blp)

    return o[:n, :out_dim]


# speedup vs baseline: 5.7680x; 1.5128x over previous
import functools

import jax
import jax.numpy as jnp
from jax import lax
from jax.experimental import pallas as pl
from jax.experimental.pallas import tpu as pltpu
from jax.experimental.pallas import tpu_sc as plsc

D = 128
CHUNK = 128
NCORES = 2
NSUB = 16
NTILES = NCORES * NSUB
HISTW = 16

_sc_mesh = plsc.VectorSubcoreMesh(core_axis_name="c", subcore_axis_name="s")


def _round_up(a, m):
    return (a + m - 1) // m * m


def _make_edge_kernel(np_, cpt):
    half = np_ // NCORES
    acc_rows = half + CHUNK
    nz = acc_rows // CHUNK
    nw = half // CHUNK
    zper = _round_up(nz, NSUB) // NSUB

    @functools.partial(
        pl.kernel,
        mesh=_sc_mesh,
        out_type=jax.ShapeDtypeStruct((NCORES, half, D), jnp.float32),
        scratch_types=[
            pltpu.VMEM((cpt, CHUNK), jnp.int32),
            pltpu.VMEM((cpt, CHUNK), jnp.int32),
            pltpu.VMEM((CHUNK, D), jnp.float32),
            pltpu.VMEM((CHUNK, D), jnp.float32),
            pltpu.SemaphoreType.DMA,
            pltpu.SemaphoreType.DMA,
            pltpu.VMEM_SHARED((acc_rows, D), jnp.float32),
        ],
    )
    def edge_kernel(y, r2d, cl2d, zrows, out, ridx, cidx, buf0, buf1, g0, g1, acc):
        cid = lax.axis_index("c")
        sid = lax.axis_index("s")
        pltpu.sync_copy(zrows, buf0)
        for z in range(zper):
            zi = z * NSUB + sid
            @pl.when(zi < nz)
            def _():
                pltpu.sync_copy(buf0, acc.at[pl.ds(zi * CHUNK, CHUNK)])
        pltpu.sync_copy(r2d.at[pl.ds(sid * cpt, cpt)], ridx)
        pltpu.sync_copy(cl2d.at[cid, pl.ds(sid * cpt, cpt)], cidx)
        plsc.subcore_barrier()

        def gather(j, buf, sem):
            return pltpu.make_async_copy(y.at[ridx.at[j]], buf, sem)

        gather(0, buf0, g0).start()
        gather(1, buf1, g1).start()

        def body(g, carry):
            j0 = 2 * g
            gather(j0, buf0, g0).wait()
            pltpu.sync_copy(buf0, acc.at[cidx.at[j0]], add=True)
            gather(j0 + 2, buf0, g0).start()
            gather(j0 + 1, buf1, g1).wait()
            pltpu.sync_copy(buf1, acc.at[cidx.at[j0 + 1]], add=True)
            gather(j0 + 3, buf1, g1).start()
            return carry

        lax.fori_loop(0, cpt // 2 - 1, body, 0)
        jlast = cpt - 2
        gather(jlast, buf0, g0).wait()
        pltpu.sync_copy(buf0, acc.at[cidx.at[jlast]], add=True)
        gather(jlast + 1, buf1, g1).wait()
        pltpu.sync_copy(buf1, acc.at[cidx.at[jlast + 1]], add=True)
        plsc.subcore_barrier()

        for z in range(zper):
            zi = z * NSUB + sid
            @pl.when(zi < nw)
            def _():
                pltpu.sync_copy(acc.at[pl.ds(zi * CHUNK, CHUNK)], buf0)
                pltpu.sync_copy(buf0, out.at[cid, pl.ds(zi * CHUNK, CHUNK)])

    return edge_kernel


def _make_deg_kernel(np_, cpt):
    half = np_ // NCORES
    acc_rows = half + CHUNK
    nz = acc_rows // CHUNK
    nw = half // CHUNK
    zper = _round_up(nz, NSUB) // NSUB

    @functools.partial(
        pl.kernel,
        mesh=_sc_mesh,
        out_type=jax.ShapeDtypeStruct((NCORES, half, D), jnp.float32),
        scratch_types=[
            pltpu.VMEM((cpt, CHUNK), jnp.int32),
            pltpu.VMEM((CHUNK, D), jnp.float32),
            pltpu.VMEM((CHUNK, D), jnp.float32),
            pltpu.VMEM_SHARED((acc_rows, D), jnp.float32),
        ],
    )
    def deg_kernel(ones_hbm, cl2d, zrows, out, cidx, ones_b, buf0, acc):
        cid = lax.axis_index("c")
        sid = lax.axis_index("s")
        pltpu.sync_copy(zrows, buf0)
        pltpu.sync_copy(ones_hbm, ones_b)
        for z in range(zper):
            zi = z * NSUB + sid
            @pl.when(zi < nz)
            def _():
                pltpu.sync_copy(buf0, acc.at[pl.ds(zi * CHUNK, CHUNK)])
        pltpu.sync_copy(cl2d.at[cid, pl.ds(sid * cpt, cpt)], cidx)
        plsc.subcore_barrier()

        def body(j, carry):
            pltpu.sync_copy(ones_b, acc.at[cidx.at[j]], add=True)
            return carry

        lax.fori_loop(0, cpt, body, 0)
        plsc.subcore_barrier()
        for z in range(zper):
            zi = z * NSUB + sid
            @pl.when(zi < nw)
            def _():
                pltpu.sync_copy(acc.at[pl.ds(zi * CHUNK, CHUNK)], buf0)
                pltpu.sync_copy(buf0, out.at[cid, pl.ds(zi * CHUNK, CHUNK)])

    return deg_kernel


def _mm_scaled_body(d_ref, x_ref, w_ref, o_ref):
    o_ref[...] = jnp.dot(d_ref[...] * x_ref[...], w_ref[...],
                         preferred_element_type=jnp.float32)


def _gcn_epilogue_mm_body(s_ref, d_ref, y_ref, b_ref, w_ref, o_ref):
    dis = d_ref[...]
    h = jax.nn.relu(dis * (s_ref[...] + 2.0 * y_ref[...]) + b_ref[...])
    o_ref[...] = jnp.dot(dis * h, w_ref[...], preferred_element_type=jnp.float32)


def _lstm_head_body(s_ref, d_ref, y_ref, b_ref, wi_ref, wt_ref, wo_ref,
                    bi_ref, bt_ref, bo_ref, wc_ref, wl_ref, bl_ref, o_ref):
    dis = d_ref[...]
    X = jax.nn.relu(dis * (s_ref[...] + 2.0 * y_ref[...]) + b_ref[...])
    for k in range(3):
        AI = jnp.dot(X, wi_ref[k], preferred_element_type=jnp.float32) + bi_ref[k]
        AT = jnp.dot(X, wt_ref[k], preferred_element_type=jnp.float32) + bt_ref[k]
        AO = jnp.dot(X, wo_ref[k], preferred_element_type=jnp.float32) + bo_ref[k]
        Cn = jax.nn.sigmoid(AI) * jnp.tanh(AT)
        X = jax.nn.sigmoid(AO + wc_ref[k] * Cn) * jnp.tanh(Cn)
    o_ref[...] = jnp.dot(jax.nn.relu(X), wl_ref[...],
                         preferred_element_type=jnp.float32) + bl_ref[...]


def kernel(x, edge_index, W1, b1, W2, b2, lstm_Wx, lstm_bx, lstm_Wh, lstm_bh,
           lstm_wc, lstm_b, Wl, bl):
    n, d = x.shape
    out_dim = Wl.shape[1]
    e = edge_index.shape[1]
    np_ = _round_up(max(n + 1, NSUB * CHUNK), NSUB * CHUNK)
    half = np_ // NCORES
    cpt_h = _round_up(e, NTILES * CHUNK) // (NTILES * CHUNK)
    cpt_h += cpt_h % 2
    e_pad = cpt_h * NTILES * CHUNK
    cpt_e = e_pad // (NSUB * CHUNK)
    blk = 512
    grid = (np_ // blk,)

    r = edge_index[0]
    c = edge_index[1]
    rp = jnp.concatenate([r, jnp.full((e_pad - e,), np_ - 1, jnp.int32)])
    cp = jnp.concatenate([c, jnp.full((e_pad - e,), n, jnp.int32)])
    r2d = rp.reshape(e_pad // CHUNK, CHUNK)
    c2d = cp.reshape(e_pad // CHUNK, CHUNK)
    trash = half + (jnp.arange(e_pad, dtype=jnp.int32) % CHUNK).reshape(c2d.shape)
    cl = []
    for core in range(NCORES):
        loc = cp.reshape(c2d.shape) - core * half
        inb = (loc >= 0) & (loc < half)
        cl.append(jnp.where(inb, loc, trash))
    cl2d = jnp.stack(cl)
    xp = jnp.zeros((np_, d), x.dtype).at[:n].set(x)
    onespat = jnp.zeros((CHUNK, HISTW), jnp.float32).at[:, 0].set(1.0)
    zhist = jnp.zeros((np_ // NSUB, HISTW), jnp.float32)
    zrows = jnp.zeros((CHUNK, d), jnp.float32)
    b1r = b1[None, :]
    b2r = b2[None, :]
    wi = lstm_Wx[:, 0]
    wt = lstm_Wx[:, 2]
    wo = lstm_Wx[:, 3]
    bi = (lstm_bx[:, 0] + lstm_bh[:, 0] + lstm_b[:, 0])[:, None, :]
    bt = (lstm_bx[:, 2] + lstm_bh[:, 2] + lstm_b[:, 2])[:, None, :]
    bo = (lstm_bx[:, 3] + lstm_bh[:, 3] + lstm_b[:, 3])[:, None, :]
    wco = lstm_wc[:, 3][:, None, :]
    wlp = jnp.zeros((d, D), jnp.float32).at[:, :out_dim].set(Wl)
    blp = jnp.zeros((1, D), jnp.float32).at[0, :out_dim].set(bl)

    edge_call = _make_edge_kernel(np_, cpt_e)

    ones_chunk = jnp.ones((CHUNK, d), jnp.float32)
    deg = _make_deg_kernel(np_, cpt_e)(
        ones_chunk, cl2d, zrows).reshape(np_, d)[:, 0] + 2.0
    dis = jnp.broadcast_to(lax.rsqrt(deg)[:, None], (np_, d))

    row_spec = pl.BlockSpec((blk, d), lambda i: (i, 0))
    w_spec = pl.BlockSpec((d, D), lambda i: (0, 0))
    b_spec = pl.BlockSpec((1, D), lambda i: (0, 0))
    w3_spec = pl.BlockSpec((3, d, d), lambda i: (0, 0, 0))
    b3_spec = pl.BlockSpec((3, 1, d), lambda i: (0, 0, 0))
    rowD = jax.ShapeDtypeStruct((np_, D), jnp.float32)

    y1 = pl.pallas_call(
        _mm_scaled_body, grid=grid,
        in_specs=[row_spec, row_spec, w_spec],
        out_specs=row_spec, out_shape=rowD,
    )(dis, xp, W1)
    seg1 = edge_call(y1, r2d, cl2d, zrows).reshape(np_, d)

    y2 = pl.pallas_call(
        _gcn_epilogue_mm_body, grid=grid,
        in_specs=[row_spec, row_spec, row_spec, b_spec, w_spec],
        out_specs=row_spec, out_shape=rowD,
    )(seg1, dis, y1, b1r, W2)
    seg2 = edge_call(y2, r2d, cl2d, zrows).reshape(np_, d)

    o = pl.pallas_call(
        _lstm_head_body, grid=grid,
        in_specs=[row_spec, row_spec, row_spec, b_spec,
                  w3_spec, w3_spec, w3_spec, b3_spec, b3_spec, b3_spec,
                  b3_spec, w_spec, b_spec],
        out_specs=row_spec, out_shape=rowD,
    )(seg2, dis, y2, b2r, wi, wt, wo, bi, bt, bo, wco, wlp, blp)

    return o[:n, :out_dim]
